# baseline (speedup 1.0000x reference)
import functools

import jax
import jax.numpy as jnp
from jax import lax
from jax.experimental import pallas as pl
from jax.experimental.pallas import tpu as pltpu
from jax.experimental.pallas import tpu_sc as plsc

_B = 53
_D = 65536
_LANES = 16
_NW = 32
_CH = 8192


@functools.lru_cache(maxsize=None)
def _sc_first_layer(nnz: int):
    nfull = nnz // _CH
    tail = nnz - nfull * _CH
    tail_iters = -(-tail // _LANES)
    chunks = [(c * _CH, _CH, _CH // _LANES) for c in range(nfull)]
    if tail:
        chunks.append((nfull * _CH, tail, tail_iters))
    mesh = plsc.VectorSubcoreMesh(core_axis_name="c", subcore_axis_name="s")

    @functools.partial(
        pl.kernel,
        mesh=mesh,
        out_type=jax.ShapeDtypeStruct((_B, 128), jnp.float32),
        scratch_types=[
            pltpu.VMEM((_D,), jnp.float32),
            pltpu.VMEM((_CH,), jnp.int32),
            pltpu.VMEM((_CH,), jnp.int32),
            pltpu.VMEM((_CH,), jnp.float32),
            pltpu.VMEM((_CH,), jnp.float32),
            pltpu.VMEM((_LANES * 64,), jnp.float32),
            pltpu.VMEM((17 * _B,), jnp.float32),
            pltpu.VMEM((32,), jnp.float32),
            pltpu.VMEM((48,), jnp.float32),
            pltpu.VMEM((16,), jnp.float32),
            pltpu.VMEM((128,), jnp.float32),
            pltpu.SemaphoreType.DMA,
            pltpu.SemaphoreType.DMA,
            pltpu.SemaphoreType.DMA,
            pltpu.SemaphoreType.DMA,
            pltpu.SemaphoreType.DMA,
        ],
        compiler_params=pltpu.CompilerParams(needs_layout_passes=False),
    )
    def sc_kernel(x_hbm, meta_hbm, vals_hbm, w2_hbm, b2_hbm, w3_hbm, b3_hbm,
                  out_hbm, xrow, meta_0, meta_1, vals_0, vals_1, acc, w2_v,
                  b2_v, w3_v, b3_v, orow, sem_x, sem_m0, sem_m1, sem_v0,
                  sem_v1):
        wid = lax.axis_index("s") * 2 + lax.axis_index("c")
        lane = lax.iota(jnp.int32, _LANES)
        zero16 = jnp.zeros((_LANES,), jnp.float32)
        pltpu.sync_copy(w2_hbm, w2_v)
        pltpu.sync_copy(b2_hbm, b2_v.at[pl.ds(0, 17)])
        pltpu.sync_copy(w3_hbm, w3_v.at[pl.ds(0, 34)])
        pltpu.sync_copy(b3_hbm, b3_v.at[pl.ds(0, 2)])
        meta_s = (meta_0, meta_1)
        vals_s = (vals_0, vals_1)
        sem_m = (sem_m0, sem_m1)
        sem_v = (sem_v0, sem_v1)

        def issue(c):
            slot = c % 2
            off, ln, n_it = chunks[c]
            if ln % _LANES:
                pad_base = (n_it - 1) * _LANES
                meta_s[slot][pl.ds(pad_base, _LANES)] = jnp.zeros(
                    (_LANES,), jnp.int32)
                vals_s[slot][pl.ds(pad_base, _LANES)] = zero16
            hm = pltpu.async_copy(meta_hbm.at[pl.ds(off, ln)],
                                  meta_s[slot].at[pl.ds(0, ln)], sem_m[slot])
            hv = pltpu.async_copy(vals_hbm.at[pl.ds(off, ln)],
                                  vals_s[slot].at[pl.ds(0, ln)], sem_v[slot])
            return hm, hv

        def do_iters(slot, n_iters):
            @plsc.parallel_loop(0, n_iters, 1, unroll=16)
            def _body(j):
                m = meta_s[slot][pl.ds(j * _LANES, _LANES)]
                v = vals_s[slot][pl.ds(j * _LANES, _LANES)]
                cidx = jnp.bitwise_and(m, 0xFFFF)
                r16 = jnp.bitwise_and(lax.shift_right_logical(m, 12), 0x3F0)
                y = plsc.load_gather(xrow, [cidx])
                plsc.addupdate_scatter(acc, [r16 + lane], y * v)

        for p in range(2):
            i = wid + p * _NW

            @pl.when(i < _B)
            def _process_row():
                hx = pltpu.async_copy(x_hbm.at[i], xrow, sem_x)
                pend = issue(0)
                for g in range(64):
                    acc[pl.ds(g * _LANES, _LANES)] = zero16
                hx.wait()
                for c in range(len(chunks)):
                    nxt = issue(c + 1) if c + 1 < len(chunks) else None
                    pend[0].wait()
                    pend[1].wait()
                    do_iters(c % 2, chunks[c][2])
                    pend = nxt
                h2v = b2_v[pl.ds(0, _LANES)]
                h2w = b2_v[pl.ds(_LANES, _LANES)]
                for k in range(_B):
                    h1k = jnp.maximum(
                        jnp.sum(acc[pl.ds(k * _LANES, _LANES)]), 0.0)
                    cola = plsc.load_gather(w2_v, [lane * _B + k])
                    colb = plsc.load_gather(
                        w2_v, [jnp.minimum(lane + 16, 16) * _B + k])
                    h2v = h2v + h1k * cola
                    h2w = h2w + h1k * colb
                h2v = jnp.maximum(h2v, 0.0)
                h2_16 = jnp.maximum(h2w[0], 0.0)
                w3r0 = w3_v[pl.ds(0, _LANES)]
                w3r1 = plsc.load_gather(w3_v, [lane + 17])
                w3hi = w3_v[pl.ds(16, _LANES)]
                w3hi2 = w3_v[pl.ds(32, _LANES)]
                b3v = b3_v[pl.ds(0, _LANES)]
                h3_0 = b3v[0] + jnp.sum(h2v * w3r0) + h2_16 * w3hi[0]
                h3_1 = b3v[1] + jnp.sum(h2v * w3r1) + h2_16 * w3hi2[1]
                mx = jnp.maximum(h3_0, h3_1)
                d = jnp.minimum(h3_0, h3_1) - mx
                dv = lax.broadcast_in_dim(d, (_LANES,), ())
                u = jnp.exp(dv)
                s = 1.0 + u
                y = u / (1.0 + 0.5 * u)
                for _ in range(3):
                    y = y - 1.0 + s * jnp.exp(-y)
                hv = jnp.where(lane == 0,
                               lax.broadcast_in_dim(h3_0, (_LANES,), ()),
                               lax.broadcast_in_dim(h3_1, (_LANES,), ()))
                ov = hv - lax.broadcast_in_dim(mx, (_LANES,), ()) - y
                orow[pl.ds(0, _LANES)] = ov
                pltpu.sync_copy(orow, out_hbm.at[i])

    return sc_kernel


def kernel(x, fc1_weight, new_vals, fc2_w, fc2_b, fc3_w, fc3_b, rows, cols):
    nnz = rows.shape[0]
    meta = jnp.bitwise_or(lax.shift_left(rows, 16), cols)
    out = _sc_first_layer(nnz)(x, meta, new_vals, fc2_w.reshape(-1), fc2_b,
                               fc3_w.reshape(-1), fc3_b)
    return out[:, :2]


# speedup vs baseline: 1.1754x; 1.1754x over previous
import functools

import jax
import jax.numpy as jnp
from jax import lax
from jax.experimental import pallas as pl
from jax.experimental.pallas import tpu as pltpu
from jax.experimental.pallas import tpu_sc as plsc

_B = 53
_D = 65536
_LANES = 16
_NW = 32
_CH = 8192


@functools.lru_cache(maxsize=None)
def _sc_first_layer(nnz: int):
    nfull = nnz // _CH
    tail = nnz - nfull * _CH
    tail_iters = -(-tail // _LANES)
    chunks = [(c * _CH, _CH, _CH // _LANES) for c in range(nfull)]
    if tail:
        chunks.append((nfull * _CH, tail, tail_iters))
    mesh = plsc.VectorSubcoreMesh(core_axis_name="c", subcore_axis_name="s")

    @functools.partial(
        pl.kernel,
        mesh=mesh,
        out_type=jax.ShapeDtypeStruct((_B, _LANES * 64), jnp.float32),
        scratch_types=[
            pltpu.VMEM((_D,), jnp.float32),
            pltpu.VMEM((_CH,), jnp.int32),
            pltpu.VMEM((_CH,), jnp.int32),
            pltpu.VMEM((_CH,), jnp.float32),
            pltpu.VMEM((_CH,), jnp.float32),
            pltpu.VMEM((_LANES * 64,), jnp.float32),
            pltpu.VMEM_SHARED((nnz,), jnp.int32),
            pltpu.VMEM_SHARED((nnz,), jnp.float32),
            pltpu.SemaphoreType.DMA,
            pltpu.SemaphoreType.DMA,
            pltpu.SemaphoreType.DMA,
            pltpu.SemaphoreType.DMA,
            pltpu.SemaphoreType.DMA,
        ],
        compiler_params=pltpu.CompilerParams(needs_layout_passes=False),
    )
    def sc_kernel(x_hbm, meta_hbm, vals_hbm, out_hbm, xrow, meta_0, meta_1,
                  vals_0, vals_1, acc, meta_sh, vals_sh, sem_x, sem_m0,
                  sem_m1, sem_v0, sem_v1):
        sid = lax.axis_index("s")
        wid = sid * 2 + lax.axis_index("c")
        lane = lax.iota(jnp.int32, _LANES)
        zero16 = jnp.zeros((_LANES,), jnp.float32)
        meta_s = (meta_0, meta_1)
        vals_s = (vals_0, vals_1)
        sem_m = (sem_m0, sem_m1)
        sem_v = (sem_v0, sem_v1)

        def issue(c):
            slot = c % 2
            off, ln, n_it = chunks[c]
            if ln % _LANES:
                pad_base = (n_it - 1) * _LANES
                meta_s[slot][pl.ds(pad_base, _LANES)] = jnp.zeros(
                    (_LANES,), jnp.int32)
                vals_s[slot][pl.ds(pad_base, _LANES)] = zero16
            hm = pltpu.async_copy(meta_sh.at[pl.ds(off, ln)],
                                  meta_s[slot].at[pl.ds(0, ln)], sem_m[slot])
            hv = pltpu.async_copy(vals_sh.at[pl.ds(off, ln)],
                                  vals_s[slot].at[pl.ds(0, ln)], sem_v[slot])
            return hm, hv

        def do_iters(slot, n_iters):
            @plsc.parallel_loop(0, n_iters, 1, unroll=16)
            def _body(j):
                m = meta_s[slot][pl.ds(j * _LANES, _LANES)]
                v = vals_s[slot][pl.ds(j * _LANES, _LANES)]
                cidx = jnp.bitwise_and(m, 0xFFFF)
                r16 = jnp.bitwise_and(lax.shift_right_logical(m, 12), 0x3F0)
                y = plsc.load_gather(xrow, [cidx])
                plsc.addupdate_scatter(acc, [r16 + lane], y * v)

        def run_pass(i, hx):
            pend = issue(0)
            for g in range(64):
                acc[pl.ds(g * _LANES, _LANES)] = zero16
            hx.wait()
            for c in range(len(chunks)):
                nxt = issue(c + 1) if c + 1 < len(chunks) else None
                pend[0].wait()
                pend[1].wait()
                do_iters(c % 2, chunks[c][2])
                pend = nxt
            pltpu.sync_copy(acc, out_hbm.at[i])

        hx0 = pltpu.async_copy(x_hbm.at[wid], xrow, sem_x)

        @pl.when(sid == 0)
        def _stage_meta():
            pltpu.sync_copy(meta_hbm, meta_sh)

        @pl.when(sid == 1)
        def _stage_vals():
            pltpu.sync_copy(vals_hbm, vals_sh)

        plsc.subcore_barrier()
        run_pass(wid, hx0)

        i1 = wid + _NW

        @pl.when(i1 < _B)
        def _process_row():
            hx1 = pltpu.async_copy(x_hbm.at[i1], xrow, sem_x)
            run_pass(i1, hx1)

    return sc_kernel


def _mlp_tail(h_ref, w2_ref, b2_ref, w3_ref, b3_ref, o_ref):
    jj = lax.broadcasted_iota(jnp.int32, (_LANES * 64, 64), 0)
    kk = lax.broadcasted_iota(jnp.int32, (_LANES * 64, 64), 1)
    fold = (lax.shift_right_logical(jj, 4) == kk).astype(jnp.float32)
    out1 = jnp.dot(h_ref[...], fold, preferred_element_type=jnp.float32,
                   precision=lax.Precision.HIGHEST)
    h1 = jnp.maximum(out1, 0.0)[:, :_B]
    nt = (((1,), (1,)), ((), ()))
    h2 = lax.dot_general(h1, w2_ref[...], nt,
                         preferred_element_type=jnp.float32)
    h2 = jnp.maximum(h2 + jnp.reshape(b2_ref[...], (1, -1)), 0.0)
    h3 = lax.dot_general(h2, w3_ref[...], nt,
                         preferred_element_type=jnp.float32)
    h3 = h3 + jnp.reshape(b3_ref[...], (1, -1))
    m = jnp.max(h3, axis=1, keepdims=True)
    e = jnp.exp(h3 - m)
    ls = jnp.log(jnp.sum(e, axis=1, keepdims=True)) + m
    o_ref[...] = h3 - ls


def kernel(x, fc1_weight, new_vals, fc2_w, fc2_b, fc3_w, fc3_b, rows, cols):
    nnz = rows.shape[0]
    meta = jnp.bitwise_or(lax.shift_left(rows, 16), cols)
    out1 = _sc_first_layer(nnz)(x, meta, new_vals)
    out = pl.pallas_call(
        _mlp_tail,
        out_shape=jax.ShapeDtypeStruct((_B, 2), jnp.float32),
    )(out1, fc2_w, fc2_b, fc3_w, fc3_b)
    return out
